# baseline (speedup 1.0000x reference)
import functools

import jax
import jax.numpy as jnp
from jax import lax
from jax.experimental import pallas as pl
from jax.experimental.pallas import tpu as pltpu
from jax.experimental.pallas import tpu_sc as plsc

NUM_CORES = 2
NUM_SUBCORES = 16
LANES = 16
NUM_WORKERS = NUM_CORES * NUM_SUBCORES

BATCH = 16384
FACTORS = 32
LINE = 128
ROWS_PER_LINE = LINE // FACTORS
B_PER_W = BATCH // NUM_WORKERS
CHUNK = 256
N_CHUNKS = B_PER_W // CHUNK


def _make_sc_kernel(inv_lines: int, td_lines: int):
  mesh = plsc.VectorSubcoreMesh(core_axis_name="c", subcore_axis_name="s")

  @functools.partial(
      pl.kernel,
      out_type=jax.ShapeDtypeStruct((BATCH,), jnp.float32),
      mesh=mesh,
      compiler_params=pltpu.CompilerParams(
          needs_layout_passes=False, use_tc_tiling_on_sc=False),
      scratch_types=[
          pltpu.VMEM((B_PER_W,), jnp.int32),
          pltpu.VMEM((B_PER_W,), jnp.int32),
          pltpu.VMEM((B_PER_W,), jnp.int32),
          pltpu.VMEM((B_PER_W,), jnp.int32),
          pltpu.VMEM((B_PER_W,), jnp.int32),
          pltpu.VMEM((B_PER_W,), jnp.int32),
          pltpu.VMEM((CHUNK, LINE), jnp.float32),
          pltpu.VMEM((CHUNK, LINE), jnp.float32),
          pltpu.VMEM((B_PER_W,), jnp.float32),
          pltpu.SemaphoreType.DMA,
          pltpu.SemaphoreType.DMA,
      ],
  )
  def dot_kernel(inv_idx_hbm, td_idx_hbm, inv_tab_hbm, td_tab_hbm, out_hbm,
                 idx_a, idx_b, line_a, line_b, colb_a, colb_b,
                 rows_a, rows_b, out_v, sem_a, sem_b):
    wid = lax.axis_index("s") * NUM_CORES + lax.axis_index("c")
    base = wid * B_PER_W

    pltpu.sync_copy(inv_idx_hbm.at[pl.ds(base, B_PER_W)], idx_a)
    pltpu.sync_copy(td_idx_hbm.at[pl.ds(base, B_PER_W)], idx_b)

    def prep_body(i):
      sl = pl.ds(i * LANES, LANES)
      va = idx_a[sl]
      vb = idx_b[sl]
      line_a[sl] = lax.shift_right_logical(va, 2)
      line_b[sl] = lax.shift_right_logical(vb, 2)
      colb_a[sl] = lax.shift_left(va & 3, 5)
      colb_b[sl] = lax.shift_left(vb & 3, 5)

    pl.loop(0, B_PER_W // LANES)(prep_body)

    lane = lax.iota(jnp.int32, LANES)

    for c in range(N_CHUNKS):
      csl = pl.ds(c * CHUNK, CHUNK)
      cp_a = pltpu.async_copy(inv_tab_hbm.at[line_a.at[csl]], rows_a, sem_a)
      cp_b = pltpu.async_copy(td_tab_hbm.at[line_b.at[csl]], rows_b, sem_b)
      cp_a.wait()
      cp_b.wait()

      def group_body(g, c=c):
        row_ids = g * LANES + lane
        gsl = pl.ds(c * CHUNK + g * LANES, LANES)
        ca = colb_a[gsl]
        cb = colb_b[gsl]
        acc = jnp.zeros((LANES,), jnp.float32)
        for d in range(FACTORS):
          va = plsc.load_gather(rows_a, [row_ids, ca + d])
          vb = plsc.load_gather(rows_b, [row_ids, cb + d])
          acc = acc + va * vb
        out_v[gsl] = acc

      pl.loop(0, CHUNK // LANES)(group_body)

    pltpu.sync_copy(out_v, out_hbm.at[pl.ds(base, B_PER_W)])

  return dot_kernel


_sc_dot = _make_sc_kernel(100000 * FACTORS // LINE, 1000000 * FACTORS // LINE)


@jax.jit
def kernel(investor, ticker, date, ticker_date, investor_factors,
           ticker_date_factors):
  del ticker, date
  inv_idx = investor.astype(jnp.int32)
  td_idx = ticker_date.astype(jnp.int32)
  n_inv, f = investor_factors.shape
  n_td, _ = ticker_date_factors.shape
  inv_lines = investor_factors.reshape(n_inv * f // LINE, LINE)
  td_lines = ticker_date_factors.reshape(n_td * f // LINE, LINE)
  return _sc_dot(inv_idx, td_idx, inv_lines, td_lines)
